# Initial kernel scaffold; baseline (speedup 1.0000x reference)
import jax
import jax.numpy as jnp
from jax.experimental import pallas as pl


def kernel(h, edge_index, edge_weight, W1, b1, g1, be1, a1, Wp, bp, W2, b2, g2, be2, a2):
    raise NotImplementedError("write your pallas kernel here")


# speedup vs baseline: 87.6878x; 87.6878x over previous
import functools
import math

import jax
import jax.numpy as jnp
from jax.experimental import pallas as pl
from jax.experimental.pallas import tpu as pltpu


def _prep_kernel(src_ref, dst_ref, ew_ref, a1_ref, as_ref, b0_ref):
    C = b0_ref.shape[0]
    E = src_ref.shape[1]
    iota_c = jax.lax.broadcasted_iota(jnp.int32, (C, E), 0)
    sh = (src_ref[...] == iota_c).astype(jnp.float32)
    dh = (dst_ref[...] == iota_c).astype(jnp.float32)
    ew = ew_ref[...]
    hi = jax.lax.Precision.HIGHEST
    b0w = jax.lax.dot_general(dh * ew, sh, (((1,), (1,)), ((), ())),
                              preferred_element_type=jnp.float32, precision=hi)
    b0u = jax.lax.dot_general(dh, sh, (((1,), (1,)), ((), ())),
                              preferred_element_type=jnp.float32, precision=hi)
    def norm_adj(b0):
        deg = jnp.sum(b0, axis=1, keepdims=True) + 1.0
        dinv = jax.lax.rsqrt(deg)
        off = (jnp.transpose(dinv) * b0) * dinv
        selfn = jnp.transpose(dinv * dinv)
        return jnp.concatenate([off, selfn], axis=0)

    a1_ref[...] = norm_adj(b0w)
    as_ref[...] = norm_adj(b0u)
    b0_ref[...] = b0w


def _k1_kernel(x_ref, w1_ref, b1_ref, a1m_ref, h1_ref, st_ref):
    Gb, C, F = x_ref.shape
    x = x_ref[...].reshape(Gb * C, F)
    xw = jnp.dot(x, w1_ref[...], preferred_element_type=jnp.float32)
    xw = xw.reshape(Gb, C, xw.shape[-1])
    am = a1m_ref[...]
    A = jnp.broadcast_to(am[:C, :], (Gb, C, C))
    h1 = jax.lax.dot_general(A, xw, (((2,), (1,)), ((0,), (0,))),
                             preferred_element_type=jnp.float32,
                             precision=jax.lax.Precision.HIGHEST)
    h1 = h1 + am[C:C + 1, :][:, :, None] * xw
    h1 = h1 + b1_ref[...][None]
    h1_ref[...] = h1

    @pl.when(pl.program_id(0) == 0)
    def _():
        st_ref[...] = jnp.zeros_like(st_ref)

    st_ref[0:1, :] += jnp.sum(h1, axis=(0, 1))[None]
    st_ref[1:2, :] += jnp.sum(h1 * h1, axis=(0, 1))[None]


def _k2_kernel(h1_ref, st1_ref, g1_ref, be1_ref, sc_ref, wp_ref, asm_ref,
               b0_ref, w2_ref, b2_ref, c2_ref, rank_ref, st2_ref,
               *, nnodes, kkeep):
    Gb, C, F = h1_ref.shape
    st = st1_ref[...]
    mu = st[0:1, :] * (1.0 / nnodes)
    var = st[1:2, :] * (1.0 / nnodes) - mu * mu
    rs = jax.lax.rsqrt(var + 1e-5)
    h1 = h1_ref[...]
    xn = (h1 - mu[None]) * (rs * g1_ref[...])[None] + be1_ref[...][None]
    a1 = sc_ref[0:1, 0:1]
    bp = sc_ref[0:1, 1:2]
    x = jnp.where(xn >= 0, xn, a1[None] * xn)
    hi = jax.lax.Precision.HIGHEST
    xb16 = x.astype(jnp.bfloat16).astype(jnp.float32)
    wpb = wp_ref[...].astype(jnp.bfloat16).astype(jnp.float32)
    spre = jnp.sum(xb16 * wpb[None], axis=2)
    asm = asm_ref[...]
    score = jax.lax.dot_general(spre, asm[:C, :], (((1,), (1,)), ((), ())),
                                preferred_element_type=jnp.float32,
                                precision=hi)
    score = score + asm[C:C + 1, :] * spre + bp
    ii = jax.lax.broadcasted_iota(jnp.int32, (C, C), 0)
    jj = jax.lax.broadcasted_iota(jnp.int32, (C, C), 1)
    si = score[:, :, None]
    sj = score[:, None, :]
    cmp = (sj > si) | ((sj == si) & (jj < ii)[None])
    rank = jnp.sum(cmp.astype(jnp.float32), axis=2)
    mask = (rank < kkeep).astype(jnp.float32)
    y = x * jnp.tanh(score)[:, :, None]
    inner = jax.lax.dot_general(mask, b0_ref[...], (((1,), (1,)), ((), ())),
                                preferred_element_type=jnp.float32,
                                precision=hi)
    deg2 = mask * (inner + 1.0)
    dinv2 = jnp.where(deg2 > 0,
                      jax.lax.rsqrt(jnp.where(deg2 > 0, deg2, 1.0)), 0.0)
    a2m = (dinv2[:, None, :] * b0_ref[...][None]) * dinv2[:, :, None]
    t = jnp.dot(y.reshape(Gb * C, F), w2_ref[...],
                preferred_element_type=jnp.float32)
    t = t.reshape(Gb, C, t.shape[-1])
    c2 = jax.lax.dot_general(a2m, t, (((2,), (1,)), ((0,), (0,))),
                             preferred_element_type=jnp.float32,
                             precision=hi)
    c2 = c2 + (dinv2 * dinv2)[:, :, None] * t
    c2 = c2 + b2_ref[...][None]
    c2_ref[...] = c2
    rank_ref[...] = rank

    @pl.when(pl.program_id(0) == 0)
    def _():
        st2_ref[...] = jnp.zeros_like(st2_ref)

    m3 = mask[:, :, None]
    st2_ref[0:1, :] += jnp.sum(m3 * c2, axis=(0, 1))[None]
    st2_ref[1:2, :] += jnp.sum(m3 * (c2 * c2), axis=(0, 1))[None]


def _k3_kernel(c2_ref, rank_ref, st2_ref, g2_ref, be2_ref, a2_ref,
               x2_ref, z_ref, *, npool, kkeep):
    Gb, C, F = c2_ref.shape
    st = st2_ref[...]
    mu = st[0:1, :] * (1.0 / npool)
    var = st[1:2, :] * (1.0 / npool) - mu * mu
    rs = jax.lax.rsqrt(var + 1e-5)
    c2 = c2_ref[...]
    xn = (c2 - mu[None]) * (rs * g2_ref[...])[None] + be2_ref[...][None]
    a2 = a2_ref[0:1, 0:1]
    xb = jnp.where(xn >= 0, xn, a2[None] * xn)
    rank = rank_ref[...]
    rr = jax.lax.broadcasted_iota(jnp.int32, (C, C), 0).astype(jnp.float32)
    perm = (rank[:, None, :] == rr[None]).astype(jnp.float32)
    x2_full = jax.lax.dot_general(perm, xb, (((2,), (1,)), ((0,), (0,))),
                                  preferred_element_type=jnp.float32,
                                  precision=jax.lax.Precision.HIGHEST)
    x2k = x2_full[:, :kkeep, :]
    x2_ref[0] = x2k.reshape(Gb * kkeep, F)
    z_ref[...] = jnp.sum(x2k, axis=1) * (1.0 / kkeep)


def kernel(h, edge_index, edge_weight, W1, b1, g1, be1, a1, Wp, bp, W2, b2,
           g2, be2, a2):
    B, C, F, Tp = h.shape
    G = B * Tp
    N = G * C
    E = edge_index.shape[1]
    H1d = W1.shape[1]
    H2d = W2.shape[1]
    kkeep = int(math.ceil(0.9 * C))
    Np = G * kkeep

    Gb = 40
    NB = G // Gb

    x = jnp.transpose(h, (0, 3, 1, 2)).reshape(G, C, F)
    src = edge_index[0].reshape(1, E).astype(jnp.int32)
    dst = edge_index[1].reshape(1, E).astype(jnp.int32)
    ewr = edge_weight.reshape(1, E)

    a1m, asm, b0w = pl.pallas_call(
        _prep_kernel,
        out_shape=[
            jax.ShapeDtypeStruct((C + 1, C), jnp.float32),
            jax.ShapeDtypeStruct((C + 1, C), jnp.float32),
            jax.ShapeDtypeStruct((C, C), jnp.float32),
        ],
    )(src, dst, ewr)

    f32 = jnp.float32
    seq = pltpu.CompilerParams(dimension_semantics=("arbitrary",))

    h1, st1 = pl.pallas_call(
        _k1_kernel,
        grid=(NB,),
        in_specs=[
            pl.BlockSpec((Gb, C, F), lambda i: (i, 0, 0)),
            pl.BlockSpec((F, H1d), lambda i: (0, 0)),
            pl.BlockSpec((1, H1d), lambda i: (0, 0)),
            pl.BlockSpec((C + 1, C), lambda i: (0, 0)),
        ],
        out_specs=[
            pl.BlockSpec((Gb, C, H1d), lambda i: (i, 0, 0)),
            pl.BlockSpec((2, H1d), lambda i: (0, 0)),
        ],
        out_shape=[
            jax.ShapeDtypeStruct((G, C, H1d), f32),
            jax.ShapeDtypeStruct((2, H1d), f32),
        ],
        compiler_params=seq,
    )(x, W1, b1.reshape(1, H1d), a1m)

    sc = jnp.concatenate([a1.reshape(1), bp.reshape(1)]).reshape(1, 2)
    sc = jnp.pad(sc, ((0, 0), (0, 126)))

    c2, rank, st2 = pl.pallas_call(
        functools.partial(_k2_kernel, nnodes=float(N), kkeep=float(kkeep)),
        grid=(NB,),
        in_specs=[
            pl.BlockSpec((Gb, C, H1d), lambda i: (i, 0, 0)),
            pl.BlockSpec((2, H1d), lambda i: (0, 0)),
            pl.BlockSpec((1, H1d), lambda i: (0, 0)),
            pl.BlockSpec((1, H1d), lambda i: (0, 0)),
            pl.BlockSpec((1, 128), lambda i: (0, 0)),
            pl.BlockSpec((1, H1d), lambda i: (0, 0)),
            pl.BlockSpec((C + 1, C), lambda i: (0, 0)),
            pl.BlockSpec((C, C), lambda i: (0, 0)),
            pl.BlockSpec((H1d, H2d), lambda i: (0, 0)),
            pl.BlockSpec((1, H2d), lambda i: (0, 0)),
        ],
        out_specs=[
            pl.BlockSpec((Gb, C, H2d), lambda i: (i, 0, 0)),
            pl.BlockSpec((Gb, C), lambda i: (i, 0)),
            pl.BlockSpec((2, H2d), lambda i: (0, 0)),
        ],
        out_shape=[
            jax.ShapeDtypeStruct((G, C, H2d), f32),
            jax.ShapeDtypeStruct((G, C), f32),
            jax.ShapeDtypeStruct((2, H2d), f32),
        ],
        compiler_params=seq,
    )(h1, st1, g1.reshape(1, H1d), be1.reshape(1, H1d), sc,
      Wp.reshape(1, H1d), asm, b0w, W2, b2.reshape(1, H2d))

    x2b, z = pl.pallas_call(
        functools.partial(_k3_kernel, npool=float(Np), kkeep=kkeep),
        grid=(NB,),
        in_specs=[
            pl.BlockSpec((Gb, C, H2d), lambda i: (i, 0, 0)),
            pl.BlockSpec((Gb, C), lambda i: (i, 0)),
            pl.BlockSpec((2, H2d), lambda i: (0, 0)),
            pl.BlockSpec((1, H2d), lambda i: (0, 0)),
            pl.BlockSpec((1, H2d), lambda i: (0, 0)),
            pl.BlockSpec((1, 128), lambda i: (0, 0)),
        ],
        out_specs=[
            pl.BlockSpec((1, Gb * kkeep, H2d), lambda i: (i, 0, 0)),
            pl.BlockSpec((Gb, H2d), lambda i: (i, 0)),
        ],
        out_shape=[
            jax.ShapeDtypeStruct((NB, Gb * kkeep, H2d), f32),
            jax.ShapeDtypeStruct((G, H2d), f32),
        ],
        compiler_params=seq,
    )(c2, rank, st2, g2.reshape(1, H2d), be2.reshape(1, H2d),
      jnp.pad(a2.reshape(1, 1), ((0, 0), (0, 127))))

    x2 = x2b.reshape(Np, H2d)
    z_seq = jnp.transpose(z.reshape(B, Tp, H2d), (0, 2, 1))
    return x2, z_seq
